# baseline (speedup 1.0000x reference)
import functools

import jax
import jax.numpy as jnp
from jax import lax
from jax.experimental import pallas as pl
from jax.experimental.pallas import tpu as pltpu
from jax.experimental.pallas import tpu_sc as plsc

_EPS = 1e-5


def _make_sc_gather(num_rows, rank):
    info = plsc.get_sparse_core_info()
    n_workers = info.num_cores * info.num_subcores
    rows_per_w = num_rows // n_workers
    chunk = 128
    n_chunks = rows_per_w // chunk
    mesh = plsc.VectorSubcoreMesh(core_axis_name="c", subcore_axis_name="s")

    @functools.partial(
        pl.kernel,
        mesh=mesh,
        out_type=jax.ShapeDtypeStruct((num_rows, rank), jnp.float32),
        scratch_types=[
            pltpu.VMEM((n_chunks, chunk), jnp.int32),
            pltpu.VMEM((rows_per_w, rank), jnp.float32),
            pltpu.SemaphoreType.DMA,
        ],
    )
    def gather_kernel(table_hbm, idx_hbm, out_hbm, idx_v, rows_v, sem):
        wid = lax.axis_index("s") * info.num_cores + lax.axis_index("c")
        pltpu.sync_copy(idx_hbm.at[pl.ds(wid * n_chunks, n_chunks)], idx_v)
        copies = [
            pltpu.async_copy(table_hbm.at[idx_v.at[j]],
                             rows_v.at[pl.ds(j * chunk, chunk)], sem)
            for j in range(n_chunks)
        ]
        for cp in copies:
            cp.wait()
        pltpu.sync_copy(rows_v, out_hbm.at[pl.ds(wid * rows_per_w, rows_per_w)])

    return gather_kernel


def _dense_body(g_ref, wt_ref, ep_ref, wp_ref, gamma_ref, beta_ref, o_ref):
    tok = lax.dot_general(g_ref[...].astype(jnp.bfloat16),
                          wt_ref[...].astype(jnp.bfloat16),
                          (((1,), (1,)), ((), ())),
                          preferred_element_type=jnp.float32)
    pos = lax.dot_general(ep_ref[...].astype(jnp.bfloat16),
                          wp_ref[...].astype(jnp.bfloat16),
                          (((1,), (1,)), ((), ())),
                          preferred_element_type=jnp.float32)
    x = tok + pos
    mu = jnp.mean(x, axis=-1, keepdims=True)
    xc = x - mu
    var = jnp.mean(xc * xc, axis=-1, keepdims=True)
    o_ref[...] = (xc * lax.rsqrt(var + _EPS) * gamma_ref[...]
                  + beta_ref[...])


def _dense_body_alias(g_ref, wt_ref, ep_ref, wp_ref, gamma_ref, beta_ref,
                      prev_ref, o_ref):
    del prev_ref
    _dense_body(g_ref, wt_ref, ep_ref, wp_ref, gamma_ref, beta_ref, o_ref)


def _dense_half(g, W_tok, E_pos, W_pos, gamma2d, beta2d, prev, half,
                n_rows_total, blk):
    emb, rank = W_tok.shape
    n_blk = g.shape[0] // blk
    off = half * n_blk
    in_specs = [
        pl.BlockSpec((blk, rank), lambda i: (i, 0)),
        pl.BlockSpec((emb, rank), lambda i: (0, 0)),
        pl.BlockSpec((blk, rank), lambda i: (0, 0)),
        pl.BlockSpec((emb, rank), lambda i: (0, 0)),
        pl.BlockSpec((1, emb), lambda i: (0, 0)),
        pl.BlockSpec((1, emb), lambda i: (0, 0)),
    ]
    args = [g, W_tok, E_pos, W_pos, gamma2d, beta2d]
    body = _dense_body
    io_alias = {}
    if prev is not None:
        in_specs.append(pl.BlockSpec(memory_space=pl.ANY))
        args.append(prev)
        body = _dense_body_alias
        io_alias = {6: 0}
    return pl.pallas_call(
        body,
        grid=(n_blk,),
        in_specs=in_specs,
        out_specs=pl.BlockSpec((blk, emb), lambda i: (i + off, 0)),
        out_shape=jax.ShapeDtypeStruct((n_rows_total, emb), jnp.float32),
        input_output_aliases=io_alias,
        compiler_params=pltpu.CompilerParams(
            dimension_semantics=("arbitrary",)),
    )(*args)


def kernel(src, E_tok, W_tok, E_pos, W_pos, gamma, beta):
    B, S = src.shape
    emb, rank = W_tok.shape
    n_rows = B * S
    idx2d = src.reshape(n_rows // 128, 128).astype(jnp.int32)

    half_rows = n_rows // 2
    half_idx_rows = idx2d.shape[0] // 2
    sc_gather = _make_sc_gather(half_rows, rank)
    g0 = sc_gather(E_tok, idx2d[:half_idx_rows])
    g1 = sc_gather(E_tok, idx2d[half_idx_rows:])

    blk = S
    gamma2d = gamma.reshape(1, emb)
    beta2d = beta.reshape(1, emb)
    out0 = _dense_half(g0, W_tok, E_pos, W_pos, gamma2d, beta2d,
                       prev=None, half=0, n_rows_total=n_rows, blk=blk)
    out = _dense_half(g1, W_tok, E_pos, W_pos, gamma2d, beta2d,
                      prev=out0, half=1, n_rows_total=n_rows, blk=blk)
    return out.reshape(B, S, emb)


# speedup vs baseline: 1.0558x; 1.0558x over previous
import functools

import jax
import jax.numpy as jnp
from jax import lax
from jax.experimental import pallas as pl
from jax.experimental.pallas import tpu as pltpu
from jax.experimental.pallas import tpu_sc as plsc

_EPS = 1e-5


def _make_sc_gather(num_rows, rank):
    info = plsc.get_sparse_core_info()
    n_workers = info.num_cores * info.num_subcores
    rows_per_w = num_rows // n_workers
    chunk = 128
    n_chunks = rows_per_w // chunk
    mesh = plsc.VectorSubcoreMesh(core_axis_name="c", subcore_axis_name="s")

    @functools.partial(
        pl.kernel,
        mesh=mesh,
        out_type=jax.ShapeDtypeStruct((num_rows, rank), jnp.float32),
        scratch_types=[
            pltpu.VMEM((n_chunks, chunk), jnp.int32),
            pltpu.VMEM((rows_per_w, rank), jnp.float32),
            pltpu.SemaphoreType.DMA,
            pltpu.SemaphoreType.DMA,
        ],
    )
    def gather_kernel(table_hbm, idx_hbm, out_hbm, idx_v, rows_v, sem_g,
                      sem_o):
        wid = lax.axis_index("s") * info.num_cores + lax.axis_index("c")
        base = wid * rows_per_w
        pltpu.sync_copy(idx_hbm.at[pl.ds(wid * n_chunks, n_chunks)], idx_v)
        gathers = [
            pltpu.async_copy(table_hbm.at[idx_v.at[j]],
                             rows_v.at[pl.ds(j * chunk, chunk)], sem_g)
            for j in range(n_chunks)
        ]
        outs = []
        for j in range(n_chunks):
            gathers[j].wait()
            outs.append(pltpu.async_copy(
                rows_v.at[pl.ds(j * chunk, chunk)],
                out_hbm.at[pl.ds(base + j * chunk, chunk)], sem_o))
        for cp in outs:
            cp.wait()

    return gather_kernel


def _dense_body(g_ref, wt_ref, ep_ref, wp_ref, gamma_ref, beta_ref, o_ref):
    tok = lax.dot_general(g_ref[...].astype(jnp.bfloat16),
                          wt_ref[...].astype(jnp.bfloat16),
                          (((1,), (1,)), ((), ())),
                          preferred_element_type=jnp.float32)
    pos = lax.dot_general(ep_ref[...].astype(jnp.bfloat16),
                          wp_ref[...].astype(jnp.bfloat16),
                          (((1,), (1,)), ((), ())),
                          preferred_element_type=jnp.float32)
    x = tok + pos
    mu = jnp.mean(x, axis=-1, keepdims=True)
    xc = x - mu
    var = jnp.mean(xc * xc, axis=-1, keepdims=True)
    o_ref[...] = (xc * lax.rsqrt(var + _EPS) * gamma_ref[...]
                  + beta_ref[...])


def kernel(src, E_tok, W_tok, E_pos, W_pos, gamma, beta):
    B, S = src.shape
    emb, rank = W_tok.shape
    n_rows = B * S
    idx2d = src.reshape(n_rows // 128, 128).astype(jnp.int32)

    gathered = _make_sc_gather(n_rows, rank)(E_tok, idx2d)

    blk = S
    n_blk = n_rows // blk
    out = pl.pallas_call(
        _dense_body,
        grid=(n_blk,),
        in_specs=[
            pl.BlockSpec((blk, rank), lambda i: (i, 0)),
            pl.BlockSpec((emb, rank), lambda i: (0, 0)),
            pl.BlockSpec((blk, rank), lambda i: (0, 0)),
            pl.BlockSpec((emb, W_pos.shape[1]), lambda i: (0, 0)),
            pl.BlockSpec((1, emb), lambda i: (0, 0)),
            pl.BlockSpec((1, emb), lambda i: (0, 0)),
        ],
        out_specs=pl.BlockSpec((blk, emb), lambda i: (i, 0)),
        out_shape=jax.ShapeDtypeStruct((n_rows, emb), jnp.float32),
        compiler_params=pltpu.CompilerParams(
            dimension_semantics=("arbitrary",)),
    )(gathered, W_tok, E_pos, W_pos,
      gamma.reshape(1, emb), beta.reshape(1, emb))

    return out.reshape(B, S, emb)


# speedup vs baseline: 1.0705x; 1.0139x over previous
import functools

import jax
import jax.numpy as jnp
from jax import lax
from jax.experimental import pallas as pl
from jax.experimental.pallas import tpu as pltpu
from jax.experimental.pallas import tpu_sc as plsc

_EPS = 1e-5


def _make_sc_gather(num_rows, rank):
    info = plsc.get_sparse_core_info()
    n_workers = info.num_cores * info.num_subcores
    rows_per_w = num_rows // n_workers
    chunk = 128
    n_chunks = rows_per_w // chunk
    mesh = plsc.VectorSubcoreMesh(core_axis_name="c", subcore_axis_name="s")

    @functools.partial(
        pl.kernel,
        mesh=mesh,
        out_type=jax.ShapeDtypeStruct((num_rows, rank), jnp.float32),
        scratch_types=[
            pltpu.VMEM((n_chunks, chunk), jnp.int32),
            pltpu.VMEM((rows_per_w, rank), jnp.float32),
            pltpu.SemaphoreType.DMA,
        ],
    )
    def gather_kernel(table_hbm, idx_hbm, out_hbm, idx_v, rows_v, sem):
        wid = lax.axis_index("s") * info.num_cores + lax.axis_index("c")
        base = wid * rows_per_w
        pltpu.sync_copy(idx_hbm.at[pl.ds(wid * n_chunks, n_chunks)], idx_v)
        copies = [
            pltpu.async_copy(table_hbm.at[idx_v.at[j]],
                             rows_v.at[pl.ds(j * chunk, chunk)], sem)
            for j in range(n_chunks)
        ]
        for cp in copies:
            cp.wait()
        pltpu.sync_copy(rows_v, out_hbm.at[pl.ds(base, rows_per_w)])

    return gather_kernel


def _dense_body(g_ref, wt_ref, ep_ref, wp_ref, gamma_ref, beta_ref, o_ref):
    tok = lax.dot_general(g_ref[...].astype(jnp.bfloat16),
                          wt_ref[...].astype(jnp.bfloat16),
                          (((1,), (1,)), ((), ())),
                          preferred_element_type=jnp.float32)
    pos = lax.dot_general(ep_ref[...].astype(jnp.bfloat16),
                          wp_ref[...].astype(jnp.bfloat16),
                          (((1,), (1,)), ((), ())),
                          preferred_element_type=jnp.float32)
    rep = tok.shape[0] // pos.shape[0]
    if rep > 1:
        pos = jnp.broadcast_to(pos[None], (rep,) + pos.shape).reshape(tok.shape)
    x = tok + pos
    mu = jnp.mean(x, axis=-1, keepdims=True)
    xc = x - mu
    var = jnp.mean(xc * xc, axis=-1, keepdims=True)
    o_ref[...] = (xc * lax.rsqrt(var + _EPS) * gamma_ref[...]
                  + beta_ref[...])


def kernel(src, E_tok, W_tok, E_pos, W_pos, gamma, beta):
    B, S = src.shape
    emb, rank = W_tok.shape
    n_rows = B * S
    idx2d = src.reshape(n_rows // 128, 128).astype(jnp.int32)

    gathered = _make_sc_gather(n_rows, rank)(E_tok, idx2d)

    blk = S
    n_blk = n_rows // blk
    out = pl.pallas_call(
        _dense_body,
        grid=(n_blk,),
        in_specs=[
            pl.BlockSpec((blk, rank), lambda i: (i, 0)),
            pl.BlockSpec((emb, rank), lambda i: (0, 0)),
            pl.BlockSpec((S, rank), lambda i: (0, 0)),
            pl.BlockSpec((emb, W_pos.shape[1]), lambda i: (0, 0)),
            pl.BlockSpec((1, emb), lambda i: (0, 0)),
            pl.BlockSpec((1, emb), lambda i: (0, 0)),
        ],
        out_specs=pl.BlockSpec((blk, emb), lambda i: (i, 0)),
        out_shape=jax.ShapeDtypeStruct((n_rows, emb), jnp.float32),
        compiler_params=pltpu.CompilerParams(
            dimension_semantics=("arbitrary",)),
    )(gathered, W_tok, E_pos, W_pos,
      gamma.reshape(1, emb), beta.reshape(1, emb))

    return out.reshape(B, S, emb)
